# baseline (speedup 1.0000x reference)
import functools

import jax
import jax.numpy as jnp
from jax import lax
from jax.experimental import pallas as pl
from jax.experimental.pallas import tpu as pltpu
from jax.experimental.pallas import tpu_sc as plsc

BATCH = 16384
HIST = 50
DIM = 16
NUM_ACT = 100000
NUM_WORKERS = 32
ROWS_W = BATCH // NUM_WORKERS
CR = 16
NCHUNKS = ROWS_W // CR
NBUF = 2
STAGE = NUM_ACT // 16

_mesh = plsc.VectorSubcoreMesh(core_axis_name="c", subcore_axis_name="s")


@functools.partial(
    pl.kernel,
    mesh=_mesh,
    out_type=jax.ShapeDtypeStruct((BATCH, HIST, DIM), jnp.float32),
    scratch_types=[
        pltpu.VMEM_SHARED((NUM_ACT, DIM), jnp.float32),
        pltpu.VMEM((NBUF, CR, HIST), jnp.int32),
        pltpu.VMEM((NBUF, CR, HIST, DIM), jnp.float32),
        pltpu.SemaphoreType.DMA,
        pltpu.SemaphoreType.DMA,
        pltpu.SemaphoreType.DMA,
        pltpu.SemaphoreType.DMA,
    ],
    compiler_params=pltpu.CompilerParams(use_tc_tiling_on_sc=False),
)
def _gather_rows(idx_hbm, table_hbm, out_hbm, table_sp, idx_v, rows_v,
                 g0, g1, s0, s1):
    cid = lax.axis_index("c")
    sid = lax.axis_index("s")
    wid = sid * 2 + cid
    row0 = wid * ROWS_W
    gsem = (g0, g1)
    ssem = (s0, s1)

    pltpu.sync_copy(table_hbm.at[pl.ds(sid * STAGE, STAGE)],
                    table_sp.at[pl.ds(sid * STAGE, STAGE)])
    plsc.subcore_barrier()

    def idx_load(g):
        b = g % NBUF
        pltpu.sync_copy(idx_hbm.at[pl.ds(row0 + g * CR, CR)], idx_v.at[b])

    def gathers_start(g):
        b = g % NBUF
        return [
            pltpu.async_copy(table_sp.at[idx_v.at[b, r]],
                             rows_v.at[b, r], gsem[b])
            for r in range(CR)
        ]

    def store_start(g):
        b = g % NBUF
        return pltpu.async_copy(
            rows_v.at[b], out_hbm.at[pl.ds(row0 + g * CR, CR)], ssem[b])

    idx_load(0)
    gh = {0: gathers_start(0)}
    sh = {}
    for g in range(NCHUNKS):
        if g + 1 < NCHUNKS:
            if g >= 1:
                sh[g - 1].wait()
            idx_load(g + 1)
            gh[g + 1] = gathers_start(g + 1)
        for h in gh.pop(g):
            h.wait()
        sh[g] = store_start(g)
    sh[NCHUNKS - 2].wait()
    sh[NCHUNKS - 1].wait()


def kernel(action_history, embedding_weight):
    out = _gather_rows(action_history.astype(jnp.int32), embedding_weight)
    return out.reshape(BATCH, HIST * DIM)


# speedup vs baseline: 1.0108x; 1.0108x over previous
import functools

import jax
import jax.numpy as jnp
from jax import lax
from jax.experimental import pallas as pl
from jax.experimental.pallas import tpu as pltpu
from jax.experimental.pallas import tpu_sc as plsc

BATCH = 16384
HIST = 50
DIM = 16
NUM_ACT = 100000
NUM_WORKERS = 32
ROWS_W = BATCH // NUM_WORKERS
CR = 16
IDX_CH = CR * HIST
NCHUNKS = ROWS_W // CR
NBUF = 2
STAGE = NUM_ACT // 16

_mesh = plsc.VectorSubcoreMesh(core_axis_name="c", subcore_axis_name="s")


@functools.partial(
    pl.kernel,
    mesh=_mesh,
    out_type=jax.ShapeDtypeStruct((BATCH, HIST, DIM), jnp.float32),
    scratch_types=[
        pltpu.VMEM_SHARED((NUM_ACT, DIM), jnp.float32),
        pltpu.VMEM((NBUF, IDX_CH), jnp.int32),
        pltpu.VMEM((NBUF, IDX_CH, DIM), jnp.float32),
        pltpu.SemaphoreType.DMA,
        pltpu.SemaphoreType.DMA,
        pltpu.SemaphoreType.DMA,
        pltpu.SemaphoreType.DMA,
    ],
    compiler_params=pltpu.CompilerParams(use_tc_tiling_on_sc=False),
)
def _gather_rows(idx_hbm, table_hbm, out_hbm, table_sp, idx_v, rows_v,
                 g0, g1, s0, s1):
    cid = lax.axis_index("c")
    sid = lax.axis_index("s")
    wid = sid * 2 + cid
    row0 = wid * ROWS_W
    gsem = (g0, g1)
    ssem = (s0, s1)

    pltpu.sync_copy(table_hbm.at[pl.ds(sid * STAGE, STAGE)],
                    table_sp.at[pl.ds(sid * STAGE, STAGE)])
    plsc.subcore_barrier()

    def idx_load(g):
        b = g % NBUF
        pltpu.sync_copy(idx_hbm.at[pl.ds((row0 + g * CR) * HIST, IDX_CH)],
                        idx_v.at[b])

    def gather_start(g):
        b = g % NBUF
        return pltpu.async_copy(
            table_sp.at[idx_v.at[b]], rows_v.at[b], gsem[b])

    def stores_start(g):
        b = g % NBUF
        return [
            pltpu.async_copy(rows_v.at[b, pl.ds(r * HIST, HIST)],
                             out_hbm.at[row0 + g * CR + r], ssem[b])
            for r in range(CR)
        ]

    idx_load(0)
    gh = {0: gather_start(0)}
    sh = {}
    for g in range(NCHUNKS):
        if g + 1 < NCHUNKS:
            if g >= 1:
                for h in sh.pop(g - 1):
                    h.wait()
            idx_load(g + 1)
            gh[g + 1] = gather_start(g + 1)
        gh.pop(g).wait()
        sh[g] = stores_start(g)
    for g in (NCHUNKS - 2, NCHUNKS - 1):
        for h in sh.pop(g):
            h.wait()


def kernel(action_history, embedding_weight):
    idx = action_history.reshape(-1).astype(jnp.int32)
    out = _gather_rows(idx, embedding_weight)
    return out.reshape(BATCH, HIST * DIM)


# speedup vs baseline: 2.1944x; 2.1711x over previous
import functools

import jax
import jax.numpy as jnp
from jax import lax
from jax.experimental import pallas as pl
from jax.experimental.pallas import tpu as pltpu
from jax.experimental.pallas import tpu_sc as plsc

BATCH = 16384
HIST = 50
DIM = 16
NUM_ACT = 100000
NUM_WORKERS = 32
ROWS_W = BATCH // NUM_WORKERS
CR = 8
IDX_CH = CR * HIST
NCHUNKS = ROWS_W // CR
NBUF = 2
STAGE = NUM_ACT // 16

_mesh = plsc.VectorSubcoreMesh(core_axis_name="c", subcore_axis_name="s")


@functools.partial(
    pl.kernel,
    mesh=_mesh,
    out_type=jax.ShapeDtypeStruct((BATCH, HIST * DIM), jnp.float32),
    scratch_types=[
        pltpu.VMEM_SHARED((NUM_ACT, DIM), jnp.float32),
        pltpu.VMEM((NBUF, IDX_CH), jnp.int32),
        pltpu.VMEM((NBUF, IDX_CH, DIM), jnp.float32),
        pltpu.VMEM((NBUF, CR, HIST * DIM), jnp.float32),
        pltpu.SemaphoreType.DMA,
        pltpu.SemaphoreType.DMA,
        pltpu.SemaphoreType.DMA,
        pltpu.SemaphoreType.DMA,
    ],
    compiler_params=pltpu.CompilerParams(use_tc_tiling_on_sc=False),
)
def _gather_rows(idx_hbm, table_hbm, out_hbm, table_sp, idx_v, rows_v,
                 flat_v, g0, g1, s0, s1):
    cid = lax.axis_index("c")
    sid = lax.axis_index("s")
    wid = sid * 2 + cid
    row0 = wid * ROWS_W
    gsem = (g0, g1)
    ssem = (s0, s1)

    pltpu.sync_copy(table_hbm.at[pl.ds(sid * STAGE, STAGE)],
                    table_sp.at[pl.ds(sid * STAGE, STAGE)])
    plsc.subcore_barrier()

    def idx_load(g):
        b = g % NBUF
        pltpu.sync_copy(idx_hbm.at[pl.ds((row0 + g * CR) * HIST, IDX_CH)],
                        idx_v.at[b])

    def gather_start(g):
        b = g % NBUF
        return pltpu.async_copy(
            table_sp.at[idx_v.at[b]], rows_v.at[b], gsem[b])

    def repack(g):
        b = g % NBUF

        def body(e, carry):
            for r in range(CR):
                flat_v[b, r, pl.ds(e * DIM, DIM)] = rows_v[b, r * HIST + e]
            return carry

        lax.fori_loop(0, HIST, body, 0)

    def store_start(g):
        b = g % NBUF
        return pltpu.async_copy(
            flat_v.at[b], out_hbm.at[pl.ds(row0 + g * CR, CR)], ssem[b])

    idx_load(0)
    gh = {0: gather_start(0)}
    sh = {}
    for g in range(NCHUNKS):
        if g + 1 < NCHUNKS:
            if g >= 1:
                sh.pop(g - 1).wait()
            idx_load(g + 1)
            gh[g + 1] = gather_start(g + 1)
        gh.pop(g).wait()
        repack(g)
        sh[g] = store_start(g)
    sh.pop(NCHUNKS - 2).wait()
    sh.pop(NCHUNKS - 1).wait()


def kernel(action_history, embedding_weight):
    idx = action_history.reshape(-1).astype(jnp.int32)
    return _gather_rows(idx, embedding_weight)
